# baseline (speedup 1.0000x reference)
import functools
import math

import jax
import jax.numpy as jnp
from jax import lax
from jax.experimental import pallas as pl
from jax.experimental.pallas import tpu as pltpu
from jax.experimental.pallas import tpu_sc as plsc

VOCAB = 100
EMBED = 512
B = 1024
S = 200
LANES = 16
NUM_CORES = 2
NUM_SUBCORES = 16
NW = NUM_CORES * NUM_SUBCORES
BPW = B // NW
C = 40
NSC = S // C
NT = NSC * BPW
GROUPS = EMBED // LANES
SCALE = math.sqrt(EMBED)


def _body(x_hbm, table_hbm, pe_hbm, out_hbm,
          idx_all, table_v, pe_v, o0, o1, so0, so1):
    cid = lax.axis_index("c")
    sid = lax.axis_index("s")
    wid = sid * NUM_CORES + cid
    b0 = wid * BPW

    o = (o0, o1)
    so = (so0, so1)

    pltpu.sync_copy(x_hbm.at[pl.ds(pl.multiple_of(b0 * S, 8), BPW * S)],
                    idx_all.at[pl.ds(0, BPW * S)])
    pltpu.sync_copy(table_hbm, table_v)
    pltpu.sync_copy(pe_hbm.at[pl.ds(0, C), :], pe_v)

    @plsc.parallel_loop(0, VOCAB * EMBED, LANES)
    def _(i):
        sl = pl.ds(i, LANES)
        table_v[sl] = table_v[sl] * SCALE

    def pair(i, _):
        for p in (0, 1):
            t = 2 * i + p
            sc = t // BPW
            s_base = pl.multiple_of(sc * C, 8)
            bi = lax.rem(t, BPW)
            b = b0 + bi

            @pl.when(jnp.logical_and(bi == 0, t > 0))
            def _():
                pltpu.sync_copy(pe_hbm.at[pl.ds(s_base, C), :], pe_v)

            @pl.when(t >= 2)
            def _():
                pltpu.make_async_copy(
                    o[p], out_hbm.at[0, pl.ds(0, C), :], so[p]).wait()

            i_base = bi * S + s_base

            @plsc.parallel_loop(0, C, unroll=2)
            def _(r):
                iv = idx_all[pl.ds(i_base + r, LANES)]
                row = pl.multiple_of(iv[0] * EMBED, LANES)
                for j in range(GROUPS):
                    sl = pl.ds(j * LANES, LANES)
                    o[p][r, sl] = table_v[pl.ds(row + j * LANES, LANES)] \
                        + pe_v[r, sl]

            pltpu.async_copy(o[p], out_hbm.at[b, pl.ds(s_base, C), :], so[p])
        return 0

    lax.fori_loop(0, NT // 2, pair, 0)

    for p in (0, 1):
        pltpu.make_async_copy(o[p], out_hbm.at[0, pl.ds(0, C), :], so[p]).wait()


@jax.jit
def kernel(x, table, pe):
    run = functools.partial(
        pl.kernel,
        out_type=jax.ShapeDtypeStruct((B, S, EMBED), jnp.float32),
        mesh=plsc.VectorSubcoreMesh(core_axis_name="c", subcore_axis_name="s"),
        scratch_types=[
            pltpu.VMEM((BPW * S + LANES,), jnp.int32),
            pltpu.VMEM((VOCAB * EMBED,), jnp.float32),
            pltpu.VMEM((C, EMBED), jnp.float32),
            pltpu.VMEM((C, EMBED), jnp.float32),
            pltpu.VMEM((C, EMBED), jnp.float32),
            pltpu.SemaphoreType.DMA,
            pltpu.SemaphoreType.DMA,
        ],
    )(_body)
    return run(x.reshape(B * S), table.reshape(VOCAB * EMBED), pe)


# speedup vs baseline: 1.2070x; 1.2070x over previous
import functools
import math

import jax
import jax.numpy as jnp
from jax import lax
from jax.experimental import pallas as pl
from jax.experimental.pallas import tpu as pltpu
from jax.experimental.pallas import tpu_sc as plsc

VOCAB = 100
EMBED = 512
B = 1024
S = 200
LANES = 16
NUM_CORES = 2
NUM_SUBCORES = 16
NW = NUM_CORES * NUM_SUBCORES
BPW = B // NW
C = 40
NSC = S // C
NT = NSC * BPW
GROUPS = EMBED // LANES
SCALE = math.sqrt(EMBED)


def _body(x_hbm, table_hbm, pe_hbm, out_hbm,
          idx_all, table_v, pe_v, o0, o1, so0, so1):
    cid = lax.axis_index("c")
    sid = lax.axis_index("s")
    wid = sid * NUM_CORES + cid
    b0 = wid * BPW

    o = (o0, o1)
    so = (so0, so1)

    pltpu.sync_copy(x_hbm.at[pl.ds(pl.multiple_of(b0 * S, 8), BPW * S)],
                    idx_all.at[pl.ds(0, BPW * S)])
    pltpu.sync_copy(table_hbm, table_v)
    pltpu.sync_copy(pe_hbm.at[pl.ds(0, C), :], pe_v)

    @plsc.parallel_loop(0, VOCAB * EMBED, LANES)
    def _(i):
        sl = pl.ds(i, LANES)
        table_v[sl] = table_v[sl] * SCALE

    def pair(i, _):
        for p in (0, 1):
            t = 2 * i + p
            sc = t // BPW
            s_base = pl.multiple_of(sc * C, 8)
            bi = lax.rem(t, BPW)
            b = b0 + bi

            @pl.when(jnp.logical_and(bi == 0, t > 0))
            def _():
                pltpu.sync_copy(pe_hbm.at[pl.ds(s_base, C), :], pe_v)

            @pl.when(t >= 2)
            def _():
                pltpu.make_async_copy(
                    o[p], out_hbm.at[0, pl.ds(0, C), :], so[p]).wait()

            i_base = bi * S + s_base

            @plsc.parallel_loop(0, C)
            def _(r):
                iv = idx_all[pl.ds(i_base + r, LANES)]
                row = pl.multiple_of(iv[0] * EMBED, LANES)
                for j in range(GROUPS):
                    sl = pl.ds(j * LANES, LANES)
                    o[p][r, sl] = table_v[pl.ds(row + j * LANES, LANES)] \
                        + pe_v[r, sl]

            pltpu.async_copy(o[p], out_hbm.at[b, pl.ds(s_base, C), :], so[p])
        return 0

    lax.fori_loop(0, NT // 2, pair, 0)

    for p in (0, 1):
        pltpu.make_async_copy(o[p], out_hbm.at[0, pl.ds(0, C), :], so[p]).wait()


@jax.jit
def kernel(x, table, pe):
    run = functools.partial(
        pl.kernel,
        out_type=jax.ShapeDtypeStruct((B, S, EMBED), jnp.float32),
        mesh=plsc.VectorSubcoreMesh(core_axis_name="c", subcore_axis_name="s"),
        scratch_types=[
            pltpu.VMEM((BPW * S + LANES,), jnp.int32),
            pltpu.VMEM((VOCAB * EMBED,), jnp.float32),
            pltpu.VMEM((C, EMBED), jnp.float32),
            pltpu.VMEM((C, EMBED), jnp.float32),
            pltpu.VMEM((C, EMBED), jnp.float32),
            pltpu.SemaphoreType.DMA,
            pltpu.SemaphoreType.DMA,
        ],
    )(_body)
    return run(x.reshape(B * S), table.reshape(VOCAB * EMBED), pe)
